# baseline (speedup 1.0000x reference)
import functools

import jax
import jax.numpy as jnp
from jax import lax
from jax.experimental import pallas as pl
from jax.experimental.pallas import tpu as pltpu
from jax.experimental.pallas import tpu_sc as plsc

_NC = 2
_NS = 16
_NW = _NC * _NS
_LANES = 16

_ROWS_PER_CHUNK = 2


def _row_splits(seq):
    splits = []
    off = 0
    while off < seq:
        n = min(128, seq - off)
        if seq - off > 128:
            n = 104
        splits.append((off, n))
        off += n
    return splits


def _embed_body(seq, embed, batch, idx_hbm, table_hbm, pos_hbm, out_hbm,
                idx_v, rows_v, obuf_v, pos_v, gs0, gs1, os0, os1):
    rows_per_worker = batch // _NW
    n_chunks = rows_per_worker // _ROWS_PER_CHUNK
    chunk = _ROWS_PER_CHUNK * seq
    gsem = (gs0, gs1)
    osem = (os0, os1)
    splits = _row_splits(seq)

    wid = lax.axis_index("s") * _NC + lax.axis_index("c")
    row_base = wid * rows_per_worker

    pltpu.sync_copy(pos_hbm.at[pl.ds(0, seq)], pos_v)

    def fire(k, b):
        r0 = row_base + k * _ROWS_PER_CHUNK
        pltpu.sync_copy(idx_hbm.at[pl.ds(r0, _ROWS_PER_CHUNK)], idx_v.at[b])
        for rep in range(_ROWS_PER_CHUNK):
            for off, n in splits:
                pltpu.async_copy(
                    table_hbm.at[idx_v.at[b, rep, pl.ds(off, n)]],
                    rows_v.at[b, pl.ds(rep * seq + off, n)],
                    gsem[b],
                )

    def process(k, b, wait_out):
        pltpu.make_async_copy(
            table_hbm.at[pl.ds(0, chunk)], rows_v.at[b], gsem[b]).wait()
        if wait_out:
            pltpu.make_async_copy(
                obuf_v.at[b], out_hbm.at[pl.ds(0, _ROWS_PER_CHUNK)],
                osem[b]).wait()

        def add_body(r, c2):
            for c in range(embed // _LANES):
                sl = pl.ds(c * _LANES, _LANES)
                p = pos_v[r, sl]
                for rep in range(_ROWS_PER_CHUNK):
                    obuf_v[b, rep, r, sl] = rows_v[b, rep * seq + r, sl] + p
            return c2

        lax.fori_loop(0, seq, add_body, 0, unroll=2)
        pltpu.async_copy(
            obuf_v.at[b],
            out_hbm.at[pl.ds(row_base + k * _ROWS_PER_CHUNK, _ROWS_PER_CHUNK)],
            osem[b],
        )

    fire(0, 0)
    fire(1, 1)
    process(0, 0, False)
    fire(2, 0)
    process(1, 1, False)
    fire(3, 1)

    def loop_body(j, carry):
        for b in range(2):
            k = 2 * j + 2 + b
            process(k, b, True)
            fire(k + 2, b)
        return carry

    lax.fori_loop(0, (n_chunks - 4) // 2, loop_body, 0)

    process(n_chunks - 2, 0, True)
    process(n_chunks - 1, 1, True)
    pltpu.make_async_copy(
        obuf_v.at[0], out_hbm.at[pl.ds(0, _ROWS_PER_CHUNK)], os0).wait()
    pltpu.make_async_copy(
        obuf_v.at[1], out_hbm.at[pl.ds(0, _ROWS_PER_CHUNK)], os1).wait()


def kernel(token_ids, text_table, pos_table):
    batch, seq = token_ids.shape
    vocab, embed = text_table.shape

    mesh = plsc.VectorSubcoreMesh(core_axis_name="c", subcore_axis_name="s")
    body = functools.partial(_embed_body, seq, embed, batch)
    out = pl.kernel(
        body,
        out_type=jax.ShapeDtypeStruct((batch, seq, embed), jnp.float32),
        mesh=mesh,
        scratch_types=[
            pltpu.VMEM((2, _ROWS_PER_CHUNK, seq), jnp.int32),
            pltpu.VMEM((2, _ROWS_PER_CHUNK * seq, embed), jnp.float32),
            pltpu.VMEM((2, _ROWS_PER_CHUNK, seq, embed), jnp.float32),
            pltpu.VMEM((seq, embed), jnp.float32),
            pltpu.SemaphoreType.DMA,
            pltpu.SemaphoreType.DMA,
            pltpu.SemaphoreType.DMA,
            pltpu.SemaphoreType.DMA,
        ],
        compiler_params=pltpu.CompilerParams(use_tc_tiling_on_sc=False),
        name="sc_embed_lookup",
    )(token_ids.astype(jnp.int32), text_table, pos_table)
    return out
